# baseline (speedup 1.0000x reference)
import functools

import jax
import jax.numpy as jnp
from jax import lax
from jax.experimental import pallas as pl

_A = 25200
_K = 300
_T = 20
_NC = 4
_SP = 1.0 - 0.05
_SN = 0.05 / 3.0


def _softplus(x):
    return jnp.maximum(x, 0.0) + jnp.log(1.0 + jnp.exp(-jnp.abs(x)))


def _loss_body(obj_ref, box_ref, cls_ref, tgt_ref, out_ref):
    obj = obj_ref[...].reshape(1, _A)
    bx = box_ref[...].reshape(4, _A)
    cl = cls_ref[...].reshape(4, _A)
    tgt = tgt_ref[...].reshape(_T, 5)

    b = lax.bitcast_convert_type(obj, jnp.int32)
    keyi = jnp.where(b >= 0, b, b ^ jnp.int32(0x7FFFFFFF))
    ukey = lax.bitcast_convert_type(keyi ^ jnp.int32(-2147483648), jnp.uint32)

    def kbody(i, th):
        bit = jnp.uint32(1) << (jnp.uint32(31) - i.astype(jnp.uint32))
        cand = th | bit
        cnt = jnp.sum((ukey >= cand).astype(jnp.int32))
        return jnp.where(cnt >= _K, cand, th)
    utheta = lax.fori_loop(0, 32, kbody, jnp.uint32(0))

    cnt_gt = jnp.sum((ukey > utheta).astype(jnp.int32))
    need = _K - cnt_gt
    eq = ukey == utheta
    iota = lax.broadcasted_iota(jnp.int32, (1, _A), 1)

    def fbody(i, x):
        cand = x | (jnp.int32(1) << (14 - i))
        g = jnp.sum((eq & (iota < cand)).astype(jnp.int32))
        return jnp.where(g < need, cand, x)
    x0 = lax.fori_loop(0, 15, fbody, jnp.int32(0))
    elig = (ukey > utheta) | (eq & (iota <= x0))

    bp0 = jax.nn.sigmoid(bx[0:1, :])
    bp1 = jax.nn.sigmoid(bx[1:2, :])
    bp2 = jax.nn.sigmoid(bx[2:3, :])
    bp3 = jax.nn.sigmoid(bx[3:4, :])
    ax1 = bp0 - bp2 * 0.5
    ay1 = bp1 - bp3 * 0.5
    ax2 = bp0 + bp2 * 0.5
    ay2 = bp1 + bp3 * 0.5
    a1 = (ax2 - ax1) * (ay2 - ay1)

    tcls = [jnp.clip(tgt[t, 0].astype(jnp.int32), 0, _NC - 1) for t in range(_T)]
    tb = [[jnp.clip(tgt[t, 1 + c], 0.0, 1.0) for c in range(4)] for t in range(_T)]

    neg1 = jnp.float32(-1.0)
    chosen = []
    for t in range(_T):
        tx1 = tb[t][0] - tb[t][2] * 0.5
        ty1 = tb[t][1] - tb[t][3] * 0.5
        tx2 = tb[t][0] + tb[t][2] * 0.5
        ty2 = tb[t][1] + tb[t][3] * 0.5
        a2 = (tx2 - tx1) * (ty2 - ty1)
        iw = jnp.clip(jnp.minimum(ax2, tx2) - jnp.maximum(ax1, tx1), 0.0, None)
        ih = jnp.clip(jnp.minimum(ay2, ty2) - jnp.maximum(ay1, ty1), 0.0, None)
        inter = iw * ih
        union = a1 + a2 - inter
        iou = jnp.where(union > 0, inter / jnp.where(union > 0, union, 1.0), 0.0)
        iou_m = jnp.where(elig, iou, neg1)
        bi = jnp.max(iou_m)
        e1 = iou_m == bi
        bk = jnp.max(jnp.where(e1, keyi, jnp.int32(-2147483647 - 1)))
        e2 = e1 & (keyi == bk)
        ci = jnp.min(jnp.where(e2, iota, jnp.int32(0x7FFFFFFF)))
        chosen.append(ci)

    is_first = []
    is_last = []
    for j in range(_T):
        f = jnp.bool_(True)
        l = jnp.bool_(True)
        for j2 in range(_T):
            same = chosen[j2] == chosen[j]
            if j2 < j:
                f = f & (~same)
            if j2 > j:
                l = l & (~same)
        is_first.append(f)
        is_last.append(l)
    cnt = jnp.float32(0.0)
    for j in range(_T):
        cnt = cnt + is_first[j].astype(jnp.float32)

    m = jnp.zeros((1, _A), dtype=jnp.bool_)
    for t in range(_T):
        m = m | (iota == chosen[t])

    bt = []
    for c in range(4):
        acc = jnp.zeros((1, _A), dtype=jnp.float32)
        for t in range(_T):
            sel = (iota == chosen[t]) & is_last[t]
            acc = acc + jnp.where(sel, tb[t][c], 0.0)
        bt.append(acc)

    cm = []
    for c in range(4):
        acc = jnp.zeros((1, _A), dtype=jnp.float32)
        for t in range(_T):
            hit = (iota == chosen[t]) & (tcls[t] == c)
            acc = acc + jnp.where(hit, 1.0, 0.0)
        cm.append(jnp.minimum(acc, 1.0))

    zf = jnp.float32(0.0)

    sp_obj = _softplus(obj)
    dense = 0.5 * jnp.sum(sp_obj)
    corr = jnp.sum(jnp.where(m, 1.5 * sp_obj - 2.0 * obj, zf))
    obj_l = (dense + corr) / _A

    cls_sum = zf
    for c in range(4):
        row = cl[c:c + 1, :]
        sp_row = _softplus(row)
        cls_sum = cls_sum + jnp.sum(jnp.where(m, sp_row, zf))
        cls_sum = cls_sum - _SN * jnp.sum(jnp.where(m, row, zf))
        cls_sum = cls_sum - (_SP - _SN) * jnp.sum(cm[c] * row)
    cls_l = cls_sum / (cnt * _NC)

    bp = [bp0, bp1, bp2, bp3]
    box_sum = zf
    for c in range(4):
        d = bp[c] - bt[c]
        ad = jnp.abs(d)
        sl1 = jnp.where(ad < 0.1, 0.5 * d * d / 0.1, ad - 0.05)
        box_sum = box_sum + jnp.sum(jnp.where(m, sl1, zf))
    box_l = box_sum / (cnt * 4.0) * 2.0

    lane = lax.broadcasted_iota(jnp.int32, (1, 128), 1)
    vec = jnp.where(lane == 0, box_l,
          jnp.where(lane == 1, cls_l,
          jnp.where(lane == 2, obj_l, zf)))
    out_ref[...] = vec.reshape(out_ref.shape)


@functools.partial(jax.jit, static_argnames=())
def kernel(predictions, targets):
    B = predictions.shape[0]
    pred = predictions.reshape(B, _A, 9)
    obj = pred[:, :, 4]
    boxT = jnp.transpose(pred[:, :, 0:4], (0, 2, 1))
    clsT = jnp.transpose(pred[:, :, 5:9], (0, 2, 1))

    out = pl.pallas_call(
        _loss_body,
        grid=(B,),
        in_specs=[
            pl.BlockSpec((1, 1, _A), lambda i: (i, 0, 0)),
            pl.BlockSpec((1, 4, _A), lambda i: (i, 0, 0)),
            pl.BlockSpec((1, 4, _A), lambda i: (i, 0, 0)),
            pl.BlockSpec((1, _T, 5), lambda i: (i, 0, 0)),
        ],
        out_specs=pl.BlockSpec((1, 1, 128), lambda i: (i, 0, 0)),
        out_shape=jax.ShapeDtypeStruct((B, 1, 128), jnp.float32),
    )(obj[:, None, :], boxT, clsT, targets)

    box_loss = jnp.sum(out[:, 0, 0])
    cls_loss = jnp.sum(out[:, 0, 1])
    obj_loss = jnp.sum(out[:, 0, 2])
    total = box_loss + cls_loss + obj_loss
    return (total, box_loss, cls_loss, obj_loss)


# speedup vs baseline: 2.8163x; 2.8163x over previous
import functools

import jax
import jax.numpy as jnp
from jax import lax
from jax.experimental import pallas as pl
from jax.experimental.pallas import tpu as pltpu
from jax.experimental.pallas import tpu_sc as plsc

_A = 25200
_K = 300
_T = 20
_NCL = 4
_SPOS = 1.0 - 0.05
_SNEG = 0.05 / 3.0
_NCAND = 4096
_NV = _A // 16
_NSEL = 320
_NF = 9
_MIN32 = -2147483648
_M31 = 0x7FFFFFFF


def _key(x):
    b = lax.bitcast_convert_type(x, jnp.int32)
    return jnp.where(b >= 0, b, b ^ _M31)


def _popcnt(m):
    return plsc.all_reduce_population_count(m)


def _sc_body(obj_hbm, predf_hbm, fld_hbm, selk_hbm, seli_hbm,
             obj_v, ckey_v, cidx_v, selk_v, seli_v, gidx_v, fld_v, smem, sem):
    nc = 2
    wid = lax.axis_index("s") * nc + lax.axis_index("c")

    @pl.when(wid < 16)
    def _worker():
        s = wid
        pltpu.sync_copy(obj_hbm.at[pl.ds(s * _A, _A)], obj_v)

        ones16 = jnp.ones((16,), jnp.int32)
        zeros16 = jnp.zeros((16,), jnp.int32)
        iot = lax.iota(jnp.int32, 16)

        def p1(i, carry):
            cntp, cnta, cntb = carry
            kk = _key(obj_v[pl.ds(i * 16, 16)])
            cntp = cntp + _popcnt(kk >= 0)
            cnta = cnta + _popcnt(kk >= jnp.int32(0x40000000))
            cntb = cntb + _popcnt(kk >= jnp.int32(-0x40000000))
            return cntp, cnta, cntb
        cntp, cnta, cntb = lax.fori_loop(0, _NV, p1, (zeros16, zeros16, zeros16))
        cnt31 = cntp[0]

        theta0 = jnp.where(cnt31 >= _K, _MIN32, jnp.int32(0))
        cnt0 = jnp.where(cnt31 >= _K, cnt31, jnp.int32(_A))
        cnt1 = jnp.where(cnt31 >= _K, cnta[0], cntb[0])
        take1 = cnt1 >= _K
        smem[0] = jnp.where(take1, theta0 | jnp.int32(1 << 30), theta0)
        smem[1] = jnp.int32(0)
        smem[2] = jnp.where(take1, cnt1, cnt0)

        def _count_pass(cand_i):
            def cp(i, acc):
                kk = _key(obj_v[pl.ds(i * 16, 16)])
                return acc + _popcnt(kk >= cand_i)
            return lax.fori_loop(0, _NV, cp, zeros16)[0]

        def _compact_pass():
            thr = smem[0] ^ _MIN32
            limit = jnp.full((16,), _NCAND, jnp.int32)
            minpad = jnp.full((16,), _MIN32, jnp.int32)
            def kp(i, off_v):
                kk = _key(obj_v[pl.ds(i * 16, 16)])
                msk = (kk >= thr) & (off_v < limit)
                csum = plsc.cumsum(jnp.where(msk, ones16, zeros16))
                pos = off_v + csum - 1
                plsc.store_scatter(ckey_v, [pos], kk, mask=msk)
                plsc.store_scatter(cidx_v, [pos], i * 16 + iot, mask=msk)
                return off_v + _popcnt(msk)
            off_v = lax.fori_loop(0, _NV, kp, zeros16)
            off = jnp.minimum(off_v[0], _NCAND)
            plsc.store_scatter(ckey_v, [off + iot], minpad)
            smem[3] = off
            smem[1] = jnp.int32(1)

        @pl.when(smem[2] <= _NCAND - 16)
        def _compact_now():
            _compact_pass()

        for lvl in range(2, 9):
            bit = jnp.int32(1 << (31 - lvl))

            @pl.when(smem[1] == 0)
            def _lvl():
                theta = smem[0]
                cnt = _count_pass((theta | bit) ^ _MIN32)
                take = cnt >= _K
                smem[0] = jnp.where(take, theta | bit, theta)
                smem[2] = jnp.where(take, cnt, smem[2])

            @pl.when((smem[1] == 0) & (smem[2] <= _NCAND - 16))
            def _do_compact():
                _compact_pass()

        @pl.when(smem[1] == 0)
        def _force_compact():
            _compact_pass()

        ncand = smem[3]
        nvec = (ncand + 15) // 16

        def bdesc(b, th):
            cand = th | (jnp.int32(1) << (jnp.int32(31) - b))
            cand_i = cand ^ _MIN32
            def cp(i, acc):
                return acc + _popcnt(ckey_v[pl.ds(i * 16, 16)] >= cand_i)
            cnt = lax.fori_loop(0, nvec, cp, zeros16)[0]
            return jnp.where(cnt >= _K, cand, th)
        theta2i = lax.fori_loop(0, 32, bdesc, jnp.int32(0)) ^ _MIN32

        kvec = jnp.full((16,), _K, jnp.int32)
        def selgt(i, off_v):
            kk = ckey_v[pl.ds(i * 16, 16)]
            ci = cidx_v[pl.ds(i * 16, 16)]
            msk = kk > theta2i
            csum = plsc.cumsum(jnp.where(msk, ones16, zeros16))
            pos = off_v + csum - 1
            plsc.store_scatter(selk_v, [pos], kk, mask=msk)
            plsc.store_scatter(seli_v, [pos], ci, mask=msk)
            return off_v + _popcnt(msk)
        c1_v = lax.fori_loop(0, nvec, selgt, zeros16)

        def seleq(i, off_v):
            kk = ckey_v[pl.ds(i * 16, 16)]
            ci = cidx_v[pl.ds(i * 16, 16)]
            msk = kk == theta2i
            csum = plsc.cumsum(jnp.where(msk, ones16, zeros16))
            pos = off_v + csum - 1
            wmsk = msk & (pos < kvec)
            plsc.store_scatter(selk_v, [pos], kk, mask=wmsk)
            plsc.store_scatter(seli_v, [pos], ci, mask=wmsk)
            return off_v + _popcnt(msk)
        lax.fori_loop(0, nvec, seleq, c1_v)

        pad = jnp.zeros((16,), jnp.int32)
        seli_v[pl.ds(304, 16)] = pad
        plsc.store_scatter(seli_v, [jnp.int32(_K) + iot], pad)
        selk_v[pl.ds(304, 16)] = pad
        plsc.store_scatter(selk_v, [jnp.int32(_K) + iot], pad)

        rowbase = jnp.int32(s * _A)
        for v in range(_NSEL // 16):
            base9 = (seli_v[pl.ds(v * 16, 16)] + rowbase) * 9
            for f in range(_NF):
                gidx_v[pl.ds(f * _NSEL + v * 16, 16)] = base9 + f
        cps = []
        for k in range(_NF * _NSEL // 96):
            cps.append(pltpu.async_copy(
                predf_hbm.at[gidx_v.at[pl.ds(k * 96, 96)]],
                fld_v.at[pl.ds(k * 96, 96)], sem))
        for cp in cps:
            cp.wait()

        pltpu.sync_copy(fld_v, fld_hbm.at[pl.ds(s * (_NF * _NSEL), _NF * _NSEL)])
        pltpu.sync_copy(selk_v.at[pl.ds(0, _NSEL)], selk_hbm.at[pl.ds(s * _NSEL, _NSEL)])
        pltpu.sync_copy(seli_v.at[pl.ds(0, _NSEL)], seli_hbm.at[pl.ds(s * _NSEL, _NSEL)])


def _softplus_tc(x):
    return jnp.maximum(x, 0.0) + jnp.log(1.0 + jnp.exp(-jnp.abs(x)))


def _tc_body(obj_ref, fld_ref, selk_ref, seli_ref, tgt_ref, out_ref):
    obj = obj_ref[...].reshape(1, _A)
    fld = fld_ref[...].reshape(_NF, _NSEL)
    selk = selk_ref[...].reshape(1, _NSEL)
    seli = seli_ref[...].reshape(1, _NSEL)
    tgt = tgt_ref[...].reshape(_T, 5)

    lane = lax.broadcasted_iota(jnp.int32, (1, _NSEL), 1)
    act = lane < _K

    bp0 = jax.nn.sigmoid(fld[0:1, :])
    bp1 = jax.nn.sigmoid(fld[1:2, :])
    bp2 = jax.nn.sigmoid(fld[2:3, :])
    bp3 = jax.nn.sigmoid(fld[3:4, :])
    ax1 = bp0 - bp2 * 0.5
    ay1 = bp1 - bp3 * 0.5
    ax2 = bp0 + bp2 * 0.5
    ay2 = bp1 + bp3 * 0.5
    a1 = (ax2 - ax1) * (ay2 - ay1)

    tcls = [jnp.clip(tgt[t, 0].astype(jnp.int32), 0, _NCL - 1) for t in range(_T)]
    tb = [[jnp.clip(tgt[t, 1 + c], 0.0, 1.0) for c in range(4)] for t in range(_T)]

    zf = jnp.float32(0.0)
    big = jnp.int32(0x7FFFFFFF)
    chosen_a = []
    chosen_p = []
    for t in range(_T):
        tx1 = tb[t][0] - tb[t][2] * 0.5
        ty1 = tb[t][1] - tb[t][3] * 0.5
        tx2 = tb[t][0] + tb[t][2] * 0.5
        ty2 = tb[t][1] + tb[t][3] * 0.5
        a2 = (tx2 - tx1) * (ty2 - ty1)
        iw = jnp.clip(jnp.minimum(ax2, tx2) - jnp.maximum(ax1, tx1), 0.0, None)
        ih = jnp.clip(jnp.minimum(ay2, ty2) - jnp.maximum(ay1, ty1), 0.0, None)
        inter = iw * ih
        union = a1 + a2 - inter
        iou = jnp.where(union > 0, inter / jnp.where(union > 0, union, 1.0), 0.0)
        iou_m = jnp.where(act, iou, jnp.float32(-1.0))
        bi = jnp.max(iou_m)
        e1 = iou_m == bi
        bk = jnp.max(jnp.where(e1, selk, jnp.int32(_MIN32)))
        e2 = e1 & (selk == bk)
        a_t = jnp.min(jnp.where(e2, seli, big))
        p_t = jnp.min(jnp.where(e2 & (seli == a_t), lane, big))
        chosen_a.append(a_t)
        chosen_p.append(p_t)

    is_first = []
    is_last = []
    for j in range(_T):
        f = jnp.bool_(True)
        l = jnp.bool_(True)
        for j2 in range(_T):
            same = chosen_a[j2] == chosen_a[j]
            if j2 < j:
                f = f & (~same)
            if j2 > j:
                l = l & (~same)
        is_first.append(f)
        is_last.append(l)
    cnt = jnp.float32(0.0)
    for j in range(_T):
        cnt = cnt + is_first[j].astype(jnp.float32)

    m = jnp.zeros((1, _NSEL), dtype=jnp.bool_)
    for t in range(_T):
        m = m | (lane == chosen_p[t])

    bt = []
    cm = []
    for c in range(4):
        acc = jnp.zeros((1, _NSEL), dtype=jnp.float32)
        for t in range(_T):
            acc = acc + jnp.where((lane == chosen_p[t]) & is_last[t], tb[t][c], zf)
        bt.append(acc)
        acc2 = jnp.zeros((1, _NSEL), dtype=jnp.float32)
        for t in range(_T):
            acc2 = acc2 + jnp.where((lane == chosen_p[t]) & (tcls[t] == c), 1.0, zf)
        cm.append(jnp.minimum(acc2, 1.0))

    sp_obj = _softplus_tc(obj)
    dense = 0.5 * jnp.sum(sp_obj)
    objl_sel = fld[4:5, :]
    sp_sel = _softplus_tc(objl_sel)
    corr = jnp.sum(jnp.where(m, 1.5 * sp_sel - 2.0 * objl_sel, zf))
    obj_l = (dense + corr) / _A

    cls_sum = zf
    for c in range(4):
        row = fld[5 + c:6 + c, :]
        sp_row = _softplus_tc(row)
        cls_sum = cls_sum + jnp.sum(jnp.where(m, sp_row, zf))
        cls_sum = cls_sum - _SNEG * jnp.sum(jnp.where(m, row, zf))
        cls_sum = cls_sum - (_SPOS - _SNEG) * jnp.sum(cm[c] * row)
    cls_l = cls_sum / (cnt * _NCL)

    bp = [bp0, bp1, bp2, bp3]
    box_sum = zf
    for c in range(4):
        d = bp[c] - bt[c]
        ad = jnp.abs(d)
        sl1 = jnp.where(ad < 0.1, 0.5 * d * d / 0.1, ad - 0.05)
        box_sum = box_sum + jnp.sum(jnp.where(m, sl1, zf))
    box_l = box_sum / (cnt * 4.0) * 2.0

    olane = lax.broadcasted_iota(jnp.int32, (1, 128), 1)
    vec = jnp.where(olane == 0, box_l,
          jnp.where(olane == 1, cls_l,
          jnp.where(olane == 2, obj_l, zf)))
    out_ref[...] = vec.reshape(out_ref.shape)


@jax.jit
def kernel(predictions, targets):
    B = predictions.shape[0]
    pred = predictions.reshape(B, _A, 9)
    obj = pred[:, :, 4]
    obj1d = pred[:, :, 4].reshape(B * _A)
    predf = predictions.reshape(B * _A * 9)

    mesh = plsc.VectorSubcoreMesh(core_axis_name="c", subcore_axis_name="s")
    f32 = jnp.float32
    i32 = jnp.int32

    @functools.partial(
        pl.kernel, mesh=mesh,
        out_type=(jax.ShapeDtypeStruct((B * _NF * _NSEL,), f32),
                  jax.ShapeDtypeStruct((B * _NSEL,), i32),
                  jax.ShapeDtypeStruct((B * _NSEL,), i32)),
        compiler_params=pltpu.CompilerParams(needs_layout_passes=False),
        scratch_types=[
            pltpu.VMEM((_A,), f32),
            pltpu.VMEM((_NCAND + 32,), i32),
            pltpu.VMEM((_NCAND + 32,), i32),
            pltpu.VMEM((_NSEL + 32,), i32),
            pltpu.VMEM((_NSEL + 32,), i32),
            pltpu.VMEM((_NF * _NSEL,), i32),
            pltpu.VMEM((_NF * _NSEL,), f32),
            pltpu.SMEM((8,), i32),
            pltpu.SemaphoreType.DMA,
        ],
    )
    def run(obj_hbm, predf_hbm, fld_hbm, selk_hbm, seli_hbm, *scratch):
        _sc_body(obj_hbm, predf_hbm, fld_hbm, selk_hbm, seli_hbm, *scratch)

    fld, selk, seli = run(obj1d, predf)

    out = pl.pallas_call(
        _tc_body,
        grid=(B,),
        in_specs=[
            pl.BlockSpec((1, 1, _A), lambda i: (i, 0, 0)),
            pl.BlockSpec((1, _NF, _NSEL), lambda i: (i, 0, 0)),
            pl.BlockSpec((1, 1, _NSEL), lambda i: (i, 0, 0)),
            pl.BlockSpec((1, 1, _NSEL), lambda i: (i, 0, 0)),
            pl.BlockSpec((1, _T, 5), lambda i: (i, 0, 0)),
        ],
        out_specs=pl.BlockSpec((1, 1, 128), lambda i: (i, 0, 0)),
        out_shape=jax.ShapeDtypeStruct((B, 1, 128), jnp.float32),
    )(obj[:, None, :], fld.reshape(B, _NF, _NSEL), selk.reshape(B, 1, _NSEL),
      seli.reshape(B, 1, _NSEL), targets)

    box_loss = jnp.sum(out[:, 0, 0])
    cls_loss = jnp.sum(out[:, 0, 1])
    obj_loss = jnp.sum(out[:, 0, 2])
    total = box_loss + cls_loss + obj_loss
    return (total, box_loss, cls_loss, obj_loss)


# speedup vs baseline: 3.1298x; 1.1113x over previous
import functools

import jax
import jax.numpy as jnp
from jax import lax
from jax.experimental import pallas as pl
from jax.experimental.pallas import tpu as pltpu
from jax.experimental.pallas import tpu_sc as plsc

_A = 25200
_K = 300
_T = 20
_NCL = 4
_SPOS = 1.0 - 0.05
_SNEG = 0.05 / 3.0
_NCAND = 4096
_NV = _A // 16
_NSEL = 320
_NF = 9
_MIN32 = -2147483648
_M31 = 0x7FFFFFFF


def _key(x):
    b = lax.bitcast_convert_type(x, jnp.int32)
    return jnp.where(b >= 0, b, b ^ _M31)


def _popcnt(m):
    return plsc.all_reduce_population_count(m)


def _sc_body(obj_hbm, predf_hbm, fld_hbm, selk_hbm, seli_hbm,
             obj_v, ckey_v, cidx_v, selk_v, seli_v, gidx_v, fld_v, smem, sem):
    nc = 2
    wid = lax.axis_index("s") * nc + lax.axis_index("c")

    @pl.when(wid < 16)
    def _worker():
        s = wid
        pltpu.sync_copy(obj_hbm.at[pl.ds(s * _A, _A)], obj_v)

        ones16 = jnp.ones((16,), jnp.int32)
        zeros16 = jnp.zeros((16,), jnp.int32)
        iot = lax.iota(jnp.int32, 16)

        def p1(i, carry):
            cntp, cnta, cntb = carry
            kk = _key(obj_v[pl.ds(i * 16, 16)])
            cntp = cntp + _popcnt(kk >= 0)
            cnta = cnta + _popcnt(kk >= jnp.int32(0x40000000))
            cntb = cntb + _popcnt(kk >= jnp.int32(-0x40000000))
            return cntp, cnta, cntb
        cntp, cnta, cntb = lax.fori_loop(0, _NV, p1, (zeros16, zeros16, zeros16))
        cnt31 = cntp[0]

        theta0 = jnp.where(cnt31 >= _K, _MIN32, jnp.int32(0))
        cnt0 = jnp.where(cnt31 >= _K, cnt31, jnp.int32(_A))
        cnt1 = jnp.where(cnt31 >= _K, cnta[0], cntb[0])
        take1 = cnt1 >= _K
        smem[0] = jnp.where(take1, theta0 | jnp.int32(1 << 30), theta0)
        smem[1] = jnp.int32(0)
        smem[2] = jnp.where(take1, cnt1, cnt0)

        def _count_pass(cand_i):
            def cp(i, acc):
                kk = _key(obj_v[pl.ds(i * 16, 16)])
                return acc + _popcnt(kk >= cand_i)
            return lax.fori_loop(0, _NV, cp, zeros16)[0]

        def _compact_pass():
            thr = smem[0] ^ _MIN32
            limit = jnp.full((16,), _NCAND, jnp.int32)
            minpad = jnp.full((16,), _MIN32, jnp.int32)
            def kp(i, off_v):
                kk = _key(obj_v[pl.ds(i * 16, 16)])
                msk = (kk >= thr) & (off_v < limit)
                csum = plsc.cumsum(jnp.where(msk, ones16, zeros16))
                pos = off_v + csum - 1
                plsc.store_scatter(ckey_v, [pos], kk, mask=msk)
                plsc.store_scatter(cidx_v, [pos], i * 16 + iot, mask=msk)
                return off_v + _popcnt(msk)
            off_v = lax.fori_loop(0, _NV, kp, zeros16)
            off = jnp.minimum(off_v[0], _NCAND)
            plsc.store_scatter(ckey_v, [off + iot], minpad)
            smem[3] = off
            smem[1] = jnp.int32(1)

        @pl.when(smem[2] <= _NCAND - 16)
        def _compact_now():
            _compact_pass()

        for lvl in range(2, 9):
            bit = jnp.int32(1 << (31 - lvl))

            @pl.when(smem[1] == 0)
            def _lvl():
                theta = smem[0]
                cnt = _count_pass((theta | bit) ^ _MIN32)
                take = cnt >= _K
                smem[0] = jnp.where(take, theta | bit, theta)
                smem[2] = jnp.where(take, cnt, smem[2])

            @pl.when((smem[1] == 0) & (smem[2] <= _NCAND - 16))
            def _do_compact():
                _compact_pass()

        @pl.when(smem[1] == 0)
        def _force_compact():
            _compact_pass()

        ncand = smem[3]
        nvec = (ncand + 15) // 16

        def bdesc(b, th):
            cand = th | (jnp.int32(1) << (jnp.int32(31) - b))
            cand_i = cand ^ _MIN32
            def cp(i, acc):
                return acc + _popcnt(ckey_v[pl.ds(i * 16, 16)] >= cand_i)
            cnt = lax.fori_loop(0, nvec, cp, zeros16)[0]
            return jnp.where(cnt >= _K, cand, th)
        theta2i = lax.fori_loop(0, 32, bdesc, jnp.int32(0)) ^ _MIN32

        kvec = jnp.full((16,), _K, jnp.int32)
        def selgt(i, off_v):
            kk = ckey_v[pl.ds(i * 16, 16)]
            ci = cidx_v[pl.ds(i * 16, 16)]
            msk = kk > theta2i
            csum = plsc.cumsum(jnp.where(msk, ones16, zeros16))
            pos = off_v + csum - 1
            plsc.store_scatter(selk_v, [pos], kk, mask=msk)
            plsc.store_scatter(seli_v, [pos], ci, mask=msk)
            return off_v + _popcnt(msk)
        c1_v = lax.fori_loop(0, nvec, selgt, zeros16)

        def seleq(i, off_v):
            kk = ckey_v[pl.ds(i * 16, 16)]
            ci = cidx_v[pl.ds(i * 16, 16)]
            msk = kk == theta2i
            csum = plsc.cumsum(jnp.where(msk, ones16, zeros16))
            pos = off_v + csum - 1
            wmsk = msk & (pos < kvec)
            plsc.store_scatter(selk_v, [pos], kk, mask=wmsk)
            plsc.store_scatter(seli_v, [pos], ci, mask=wmsk)
            return off_v + _popcnt(msk)
        lax.fori_loop(0, nvec, seleq, c1_v)

        pad = jnp.zeros((16,), jnp.int32)
        seli_v[pl.ds(304, 16)] = pad
        plsc.store_scatter(seli_v, [jnp.int32(_K) + iot], pad)
        selk_v[pl.ds(304, 16)] = pad
        plsc.store_scatter(selk_v, [jnp.int32(_K) + iot], pad)

        rowbase = jnp.int32(s * _A)
        for v in range(_NSEL // 16):
            base9 = (seli_v[pl.ds(v * 16, 16)] + rowbase) * 9
            for f in range(_NF):
                gidx_v[pl.ds(f * _NSEL + v * 16, 16)] = base9 + f
        cps = []
        for k in range(_NF * _NSEL // 96):
            cps.append(pltpu.async_copy(
                predf_hbm.at[gidx_v.at[pl.ds(k * 96, 96)]],
                fld_v.at[pl.ds(k * 96, 96)], sem))
        for cp in cps:
            cp.wait()

        pltpu.sync_copy(fld_v, fld_hbm.at[pl.ds(s * (_NF * _NSEL), _NF * _NSEL)])
        pltpu.sync_copy(selk_v.at[pl.ds(0, _NSEL)], selk_hbm.at[pl.ds(s * _NSEL, _NSEL)])
        pltpu.sync_copy(seli_v.at[pl.ds(0, _NSEL)], seli_hbm.at[pl.ds(s * _NSEL, _NSEL)])


def _softplus_tc(x):
    return jnp.maximum(x, 0.0) + jnp.log(1.0 + jnp.exp(-jnp.abs(x)))


def _tc_body(obj_ref, fld_ref, selk_ref, seli_ref, tgt_ref, out_ref):
    obj = obj_ref[...].reshape(8, _A // 8)
    fld = fld_ref[...].reshape(_NF, _NSEL)
    selk = selk_ref[...].reshape(1, _NSEL)
    seli = seli_ref[...].reshape(1, _NSEL)
    tgt = tgt_ref[...].reshape(_T, 5)

    lane = lax.broadcasted_iota(jnp.int32, (1, _NSEL), 1)
    act = lane < _K

    bp0 = jax.nn.sigmoid(fld[0:1, :])
    bp1 = jax.nn.sigmoid(fld[1:2, :])
    bp2 = jax.nn.sigmoid(fld[2:3, :])
    bp3 = jax.nn.sigmoid(fld[3:4, :])
    ax1 = bp0 - bp2 * 0.5
    ay1 = bp1 - bp3 * 0.5
    ax2 = bp0 + bp2 * 0.5
    ay2 = bp1 + bp3 * 0.5
    a1 = (ax2 - ax1) * (ay2 - ay1)

    tcls = [jnp.clip(tgt[t, 0].astype(jnp.int32), 0, _NCL - 1) for t in range(_T)]
    tb = [[jnp.clip(tgt[t, 1 + c], 0.0, 1.0) for c in range(4)] for t in range(_T)]

    zf = jnp.float32(0.0)
    big = jnp.int32(0x7FFFFFFF)
    chosen_a = []
    chosen_p = []
    for t in range(_T):
        tx1 = tb[t][0] - tb[t][2] * 0.5
        ty1 = tb[t][1] - tb[t][3] * 0.5
        tx2 = tb[t][0] + tb[t][2] * 0.5
        ty2 = tb[t][1] + tb[t][3] * 0.5
        a2 = (tx2 - tx1) * (ty2 - ty1)
        iw = jnp.clip(jnp.minimum(ax2, tx2) - jnp.maximum(ax1, tx1), 0.0, None)
        ih = jnp.clip(jnp.minimum(ay2, ty2) - jnp.maximum(ay1, ty1), 0.0, None)
        inter = iw * ih
        union = a1 + a2 - inter
        iou = jnp.where(union > 0, inter / jnp.where(union > 0, union, 1.0), 0.0)
        iou_m = jnp.where(act, iou, jnp.float32(-1.0))
        bi = jnp.max(iou_m)
        e1 = iou_m == bi
        bk = jnp.max(jnp.where(e1, selk, jnp.int32(_MIN32)))
        e2 = e1 & (selk == bk)
        a_t = jnp.min(jnp.where(e2, seli, big))
        p_t = jnp.min(jnp.where(e2 & (seli == a_t), lane, big))
        chosen_a.append(a_t)
        chosen_p.append(p_t)

    is_first = []
    is_last = []
    for j in range(_T):
        f = jnp.bool_(True)
        l = jnp.bool_(True)
        for j2 in range(_T):
            same = chosen_a[j2] == chosen_a[j]
            if j2 < j:
                f = f & (~same)
            if j2 > j:
                l = l & (~same)
        is_first.append(f)
        is_last.append(l)
    cnt = jnp.float32(0.0)
    for j in range(_T):
        cnt = cnt + is_first[j].astype(jnp.float32)

    m = jnp.zeros((1, _NSEL), dtype=jnp.bool_)
    for t in range(_T):
        m = m | (lane == chosen_p[t])

    bt = []
    cm = []
    for c in range(4):
        acc = jnp.zeros((1, _NSEL), dtype=jnp.float32)
        for t in range(_T):
            acc = acc + jnp.where((lane == chosen_p[t]) & is_last[t], tb[t][c], zf)
        bt.append(acc)
        acc2 = jnp.zeros((1, _NSEL), dtype=jnp.float32)
        for t in range(_T):
            acc2 = acc2 + jnp.where((lane == chosen_p[t]) & (tcls[t] == c), 1.0, zf)
        cm.append(jnp.minimum(acc2, 1.0))

    sp_obj = _softplus_tc(obj)
    dense = 0.5 * jnp.sum(sp_obj)
    objl_sel = fld[4:5, :]
    sp_sel = _softplus_tc(objl_sel)
    corr = jnp.sum(jnp.where(m, 1.5 * sp_sel - 2.0 * objl_sel, zf))
    obj_l = (dense + corr) / _A

    cls_sum = zf
    for c in range(4):
        row = fld[5 + c:6 + c, :]
        sp_row = _softplus_tc(row)
        cls_sum = cls_sum + jnp.sum(jnp.where(m, sp_row, zf))
        cls_sum = cls_sum - _SNEG * jnp.sum(jnp.where(m, row, zf))
        cls_sum = cls_sum - (_SPOS - _SNEG) * jnp.sum(cm[c] * row)
    cls_l = cls_sum / (cnt * _NCL)

    bp = [bp0, bp1, bp2, bp3]
    box_sum = zf
    for c in range(4):
        d = bp[c] - bt[c]
        ad = jnp.abs(d)
        sl1 = jnp.where(ad < 0.1, 0.5 * d * d / 0.1, ad - 0.05)
        box_sum = box_sum + jnp.sum(jnp.where(m, sl1, zf))
    box_l = box_sum / (cnt * 4.0) * 2.0

    olane = lax.broadcasted_iota(jnp.int32, (1, 128), 1)
    vec = jnp.where(olane == 0, box_l,
          jnp.where(olane == 1, cls_l,
          jnp.where(olane == 2, obj_l, zf)))
    out_ref[...] = vec.reshape(out_ref.shape)


@jax.jit
def kernel(predictions, targets):
    B = predictions.shape[0]
    pred = predictions.reshape(B, _A, 9)
    obj = pred[:, :, 4]
    obj1d = obj.reshape(B * _A)
    predf = predictions.reshape(B * _A * 9)

    mesh = plsc.VectorSubcoreMesh(core_axis_name="c", subcore_axis_name="s")
    f32 = jnp.float32
    i32 = jnp.int32

    @functools.partial(
        pl.kernel, mesh=mesh,
        out_type=(jax.ShapeDtypeStruct((B * _NF * _NSEL,), f32),
                  jax.ShapeDtypeStruct((B * _NSEL,), i32),
                  jax.ShapeDtypeStruct((B * _NSEL,), i32)),
        compiler_params=pltpu.CompilerParams(needs_layout_passes=False),
        scratch_types=[
            pltpu.VMEM((_A,), f32),
            pltpu.VMEM((_NCAND + 32,), i32),
            pltpu.VMEM((_NCAND + 32,), i32),
            pltpu.VMEM((_NSEL + 32,), i32),
            pltpu.VMEM((_NSEL + 32,), i32),
            pltpu.VMEM((_NF * _NSEL,), i32),
            pltpu.VMEM((_NF * _NSEL,), f32),
            pltpu.SMEM((8,), i32),
            pltpu.SemaphoreType.DMA,
        ],
    )
    def run(obj_hbm, predf_hbm, fld_hbm, selk_hbm, seli_hbm, *scratch):
        _sc_body(obj_hbm, predf_hbm, fld_hbm, selk_hbm, seli_hbm, *scratch)

    fld, selk, seli = run(obj1d, predf)

    out = pl.pallas_call(
        _tc_body,
        grid=(B,),
        in_specs=[
            pl.BlockSpec((1, 8, _A // 8), lambda i: (i, 0, 0)),
            pl.BlockSpec((1, _NF, _NSEL), lambda i: (i, 0, 0)),
            pl.BlockSpec((1, 1, _NSEL), lambda i: (i, 0, 0)),
            pl.BlockSpec((1, 1, _NSEL), lambda i: (i, 0, 0)),
            pl.BlockSpec((1, _T, 5), lambda i: (i, 0, 0)),
        ],
        out_specs=pl.BlockSpec((1, 1, 128), lambda i: (i, 0, 0)),
        out_shape=jax.ShapeDtypeStruct((B, 1, 128), jnp.float32),
    )(obj.reshape(B, 8, _A // 8), fld.reshape(B, _NF, _NSEL), selk.reshape(B, 1, _NSEL),
      seli.reshape(B, 1, _NSEL), targets)

    box_loss = jnp.sum(out[:, 0, 0])
    cls_loss = jnp.sum(out[:, 0, 1])
    obj_loss = jnp.sum(out[:, 0, 2])
    total = box_loss + cls_loss + obj_loss
    return (total, box_loss, cls_loss, obj_loss)


# speedup vs baseline: 4.3318x; 1.3841x over previous
import functools

import jax
import jax.numpy as jnp
from jax import lax
from jax.experimental import pallas as pl
from jax.experimental.pallas import tpu as pltpu
from jax.experimental.pallas import tpu_sc as plsc

_A = 25200
_K = 300
_T = 20
_NCL = 4
_SPOS = 1.0 - 0.05
_SNEG = 0.05 / 3.0
_NCAND = 4096
_NV = _A // 16
_NSEL = 320
_NF = 9
_MIN32 = -2147483648
_M31 = 0x7FFFFFFF


def _key(x):
    b = lax.bitcast_convert_type(x, jnp.int32)
    return jnp.where(b >= 0, b, b ^ _M31)


def _popcnt(m):
    return plsc.all_reduce_population_count(m)


def _sc_body(obj_hbm, predf_hbm, fld_hbm, selk_hbm, seli_hbm,
             obj_v, ckey_v, cidx_v, selk_v, seli_v, gidx_v, fld_v, smem, sem):
    nc = 2
    wid = lax.axis_index("s") * nc + lax.axis_index("c")

    @pl.when(wid < 16)
    def _worker():
        s = wid
        pltpu.sync_copy(obj_hbm.at[pl.ds(s * _A, _A)], obj_v)

        ones16 = jnp.ones((16,), jnp.int32)
        zeros16 = jnp.zeros((16,), jnp.int32)
        iot = lax.iota(jnp.int32, 16)

        def p1(i, carry):
            cntp, cnta, cntb = carry
            kk = _key(obj_v[pl.ds(i * 16, 16)])
            cntp = cntp + _popcnt(kk >= 0)
            cnta = cnta + _popcnt(kk >= jnp.int32(0x40000000))
            cntb = cntb + _popcnt(kk >= jnp.int32(-0x40000000))
            return cntp, cnta, cntb
        cntp, cnta, cntb = lax.fori_loop(0, _NV, p1, (zeros16, zeros16, zeros16))
        cnt31 = cntp[0]

        theta0 = jnp.where(cnt31 >= _K, _MIN32, jnp.int32(0))
        cnt0 = jnp.where(cnt31 >= _K, cnt31, jnp.int32(_A))
        cnt1 = jnp.where(cnt31 >= _K, cnta[0], cntb[0])
        take1 = cnt1 >= _K
        smem[0] = jnp.where(take1, theta0 | jnp.int32(1 << 30), theta0)
        smem[1] = jnp.int32(0)
        smem[2] = jnp.where(take1, cnt1, cnt0)

        def _count_pass(cand_i):
            def cp(i, acc):
                kk = _key(obj_v[pl.ds(i * 16, 16)])
                return acc + _popcnt(kk >= cand_i)
            return lax.fori_loop(0, _NV, cp, zeros16)[0]

        def _compact_pass():
            thr = smem[0] ^ _MIN32
            limit = jnp.full((16,), _NCAND, jnp.int32)
            minpad = jnp.full((16,), _MIN32, jnp.int32)
            def kp(i, off_v):
                kk = _key(obj_v[pl.ds(i * 16, 16)])
                msk = (kk >= thr) & (off_v < limit)
                csum = plsc.cumsum(jnp.where(msk, ones16, zeros16))
                pos = off_v + csum - 1
                plsc.store_scatter(ckey_v, [pos], kk, mask=msk)
                plsc.store_scatter(cidx_v, [pos], i * 16 + iot, mask=msk)
                return off_v + _popcnt(msk)
            off_v = lax.fori_loop(0, _NV, kp, zeros16)
            off = jnp.minimum(off_v[0], _NCAND)
            plsc.store_scatter(ckey_v, [off + iot], minpad)
            smem[3] = off
            smem[1] = jnp.int32(1)

        @pl.when(smem[2] <= _NCAND - 16)
        def _compact_now():
            _compact_pass()

        for lvl in range(2, 9):
            bit = jnp.int32(1 << (31 - lvl))

            @pl.when(smem[1] == 0)
            def _lvl():
                theta = smem[0]
                cnt = _count_pass((theta | bit) ^ _MIN32)
                take = cnt >= _K
                smem[0] = jnp.where(take, theta | bit, theta)
                smem[2] = jnp.where(take, cnt, smem[2])

            @pl.when((smem[1] == 0) & (smem[2] <= _NCAND - 16))
            def _do_compact():
                _compact_pass()

        @pl.when(smem[1] == 0)
        def _force_compact():
            _compact_pass()

        ncand = smem[3]
        nvec = (ncand + 15) // 16

        def bdesc(b, th):
            cand = th | (jnp.int32(1) << (jnp.int32(31) - b))
            cand_i = cand ^ _MIN32
            def cp(i, acc):
                return acc + _popcnt(ckey_v[pl.ds(i * 16, 16)] >= cand_i)
            cnt = lax.fori_loop(0, nvec, cp, zeros16)[0]
            return jnp.where(cnt >= _K, cand, th)
        theta2i = lax.fori_loop(0, 32, bdesc, jnp.int32(0)) ^ _MIN32

        kvec = jnp.full((16,), _K, jnp.int32)
        def selgt(i, off_v):
            kk = ckey_v[pl.ds(i * 16, 16)]
            ci = cidx_v[pl.ds(i * 16, 16)]
            msk = kk > theta2i
            csum = plsc.cumsum(jnp.where(msk, ones16, zeros16))
            pos = off_v + csum - 1
            plsc.store_scatter(selk_v, [pos], kk, mask=msk)
            plsc.store_scatter(seli_v, [pos], ci, mask=msk)
            return off_v + _popcnt(msk)
        c1_v = lax.fori_loop(0, nvec, selgt, zeros16)

        def seleq(i, off_v):
            kk = ckey_v[pl.ds(i * 16, 16)]
            ci = cidx_v[pl.ds(i * 16, 16)]
            msk = kk == theta2i
            csum = plsc.cumsum(jnp.where(msk, ones16, zeros16))
            pos = off_v + csum - 1
            wmsk = msk & (pos < kvec)
            plsc.store_scatter(selk_v, [pos], kk, mask=wmsk)
            plsc.store_scatter(seli_v, [pos], ci, mask=wmsk)
            return off_v + _popcnt(msk)
        lax.fori_loop(0, nvec, seleq, c1_v)

        pad = jnp.zeros((16,), jnp.int32)
        seli_v[pl.ds(304, 16)] = pad
        plsc.store_scatter(seli_v, [jnp.int32(_K) + iot], pad)
        selk_v[pl.ds(304, 16)] = pad
        plsc.store_scatter(selk_v, [jnp.int32(_K) + iot], pad)

        rowbase = jnp.int32(s * _A)
        for v in range(_NSEL // 16):
            base9 = (seli_v[pl.ds(v * 16, 16)] + rowbase) * 9
            for f in range(_NF):
                gidx_v[pl.ds(f * _NSEL + v * 16, 16)] = base9 + f
        cps = []
        for k in range(_NF * _NSEL // 96):
            cps.append(pltpu.async_copy(
                predf_hbm.at[gidx_v.at[pl.ds(k * 96, 96)]],
                fld_v.at[pl.ds(k * 96, 96)], sem))
        for cp in cps:
            cp.wait()

        pltpu.sync_copy(fld_v, fld_hbm.at[pl.ds(s * (_NF * _NSEL), _NF * _NSEL)])
        pltpu.sync_copy(selk_v.at[pl.ds(0, _NSEL)], selk_hbm.at[pl.ds(s * _NSEL, _NSEL)])
        pltpu.sync_copy(seli_v.at[pl.ds(0, _NSEL)], seli_hbm.at[pl.ds(s * _NSEL, _NSEL)])


def _softplus_tc(x):
    return jnp.maximum(x, 0.0) + jnp.log(1.0 + jnp.exp(-jnp.abs(x)))


def _tc_body(obj_ref, fld_ref, selk_ref, seli_ref, tgt_ref, out_ref):
    obj = obj_ref[...].reshape(8, _A // 8)
    fld = fld_ref[...].reshape(_NF, _NSEL)
    selk = selk_ref[...].reshape(1, _NSEL)
    seli = seli_ref[...].reshape(1, _NSEL)
    tgt = tgt_ref[...].reshape(_T, 5)

    lane = lax.broadcasted_iota(jnp.int32, (1, _NSEL), 1)
    act = lane < _K
    zf = jnp.float32(0.0)
    big = jnp.int32(0x7FFFFFFF)

    bp0 = jax.nn.sigmoid(fld[0:1, :])
    bp1 = jax.nn.sigmoid(fld[1:2, :])
    bp2 = jax.nn.sigmoid(fld[2:3, :])
    bp3 = jax.nn.sigmoid(fld[3:4, :])
    ax1 = bp0 - bp2 * 0.5
    ay1 = bp1 - bp3 * 0.5
    ax2 = bp0 + bp2 * 0.5
    ay2 = bp1 + bp3 * 0.5
    a1 = (ax2 - ax1) * (ay2 - ay1)

    tcls = jnp.clip(tgt[:, 0:1].astype(jnp.int32), 0, _NCL - 1)
    tbc = [jnp.clip(tgt[:, 1 + c:2 + c], 0.0, 1.0) for c in range(4)]
    tx1 = tbc[0] - tbc[2] * 0.5
    ty1 = tbc[1] - tbc[3] * 0.5
    tx2 = tbc[0] + tbc[2] * 0.5
    ty2 = tbc[1] + tbc[3] * 0.5
    a2 = (tx2 - tx1) * (ty2 - ty1)

    iw = jnp.clip(jnp.minimum(ax2, tx2) - jnp.maximum(ax1, tx1), 0.0, None)
    ih = jnp.clip(jnp.minimum(ay2, ty2) - jnp.maximum(ay1, ty1), 0.0, None)
    inter = iw * ih
    union = a1 + a2 - inter
    iou = jnp.where(union > 0, inter / jnp.where(union > 0, union, 1.0), 0.0)
    iou_m = jnp.where(act, iou, jnp.float32(-1.0))

    bi = jnp.max(iou_m, axis=1, keepdims=True)
    e1 = iou_m == bi
    bk = jnp.max(jnp.where(e1, selk, jnp.int32(_MIN32)), axis=1, keepdims=True)
    e2 = e1 & (selk == bk)
    a_t = jnp.min(jnp.where(e2, seli, big), axis=1, keepdims=True)
    p_t = jnp.min(jnp.where(e2 & (seli == a_t), lane, big), axis=1, keepdims=True)

    a_row = jnp.transpose(a_t)
    same = a_t == a_row
    jj_col = lax.broadcasted_iota(jnp.int32, (_T, 1), 0)
    jj_row = lax.broadcasted_iota(jnp.int32, (1, _T), 1)
    is_first = ~jnp.any(same & (jj_row < jj_col), axis=1, keepdims=True)
    is_last = ~jnp.any(same & (jj_row > jj_col), axis=1, keepdims=True)
    cnt = jnp.sum(is_first.astype(jnp.float32))

    eqp = lane == p_t
    m = jnp.any(eqp, axis=0, keepdims=True)

    sp_obj = _softplus_tc(obj)
    dense = 0.5 * jnp.sum(sp_obj)
    objl_sel = fld[4:5, :]
    sp_sel = _softplus_tc(objl_sel)
    corr = jnp.sum(jnp.where(m, 1.5 * sp_sel - 2.0 * objl_sel, zf))
    obj_l = (dense + corr) / _A

    cls_sum = zf
    for c in range(4):
        row = fld[5 + c:6 + c, :]
        sp_row = _softplus_tc(row)
        cls_sum = cls_sum + jnp.sum(jnp.where(m, sp_row, zf))
        cls_sum = cls_sum - _SNEG * jnp.sum(jnp.where(m, row, zf))
        cm_c = jnp.minimum(
            jnp.sum(jnp.where(eqp & (tcls == c), 1.0, zf), axis=0, keepdims=True), 1.0)
        cls_sum = cls_sum - (_SPOS - _SNEG) * jnp.sum(cm_c * row)
    cls_l = cls_sum / (cnt * _NCL)

    bp = [bp0, bp1, bp2, bp3]
    box_sum = zf
    for c in range(4):
        bt_c = jnp.sum(jnp.where(eqp & is_last, tbc[c], zf), axis=0, keepdims=True)
        d = bp[c] - bt_c
        ad = jnp.abs(d)
        sl1 = jnp.where(ad < 0.1, 0.5 * d * d / 0.1, ad - 0.05)
        box_sum = box_sum + jnp.sum(jnp.where(m, sl1, zf))
    box_l = box_sum / (cnt * 4.0) * 2.0

    olane = lax.broadcasted_iota(jnp.int32, (1, 128), 1)
    vec = jnp.where(olane == 0, box_l,
          jnp.where(olane == 1, cls_l,
          jnp.where(olane == 2, obj_l, zf)))
    out_ref[...] = vec.reshape(out_ref.shape)


@jax.jit
def kernel(predictions, targets):
    B = predictions.shape[0]
    pred = predictions.reshape(B, _A, 9)
    obj = pred[:, :, 4]
    obj1d = obj.reshape(B * _A)
    predf = predictions.reshape(B * _A * 9)

    mesh = plsc.VectorSubcoreMesh(core_axis_name="c", subcore_axis_name="s")
    f32 = jnp.float32
    i32 = jnp.int32

    @functools.partial(
        pl.kernel, mesh=mesh,
        out_type=(jax.ShapeDtypeStruct((B * _NF * _NSEL,), f32),
                  jax.ShapeDtypeStruct((B * _NSEL,), i32),
                  jax.ShapeDtypeStruct((B * _NSEL,), i32)),
        compiler_params=pltpu.CompilerParams(needs_layout_passes=False),
        scratch_types=[
            pltpu.VMEM((_A,), f32),
            pltpu.VMEM((_NCAND + 32,), i32),
            pltpu.VMEM((_NCAND + 32,), i32),
            pltpu.VMEM((_NSEL + 32,), i32),
            pltpu.VMEM((_NSEL + 32,), i32),
            pltpu.VMEM((_NF * _NSEL,), i32),
            pltpu.VMEM((_NF * _NSEL,), f32),
            pltpu.SMEM((8,), i32),
            pltpu.SemaphoreType.DMA,
        ],
    )
    def run(obj_hbm, predf_hbm, fld_hbm, selk_hbm, seli_hbm, *scratch):
        _sc_body(obj_hbm, predf_hbm, fld_hbm, selk_hbm, seli_hbm, *scratch)

    fld, selk, seli = run(obj1d, predf)

    out = pl.pallas_call(
        _tc_body,
        grid=(B,),
        in_specs=[
            pl.BlockSpec((1, 8, _A // 8), lambda i: (i, 0, 0)),
            pl.BlockSpec((1, _NF, _NSEL), lambda i: (i, 0, 0)),
            pl.BlockSpec((1, 1, _NSEL), lambda i: (i, 0, 0)),
            pl.BlockSpec((1, 1, _NSEL), lambda i: (i, 0, 0)),
            pl.BlockSpec((1, _T, 5), lambda i: (i, 0, 0)),
        ],
        out_specs=pl.BlockSpec((1, 1, 128), lambda i: (i, 0, 0)),
        out_shape=jax.ShapeDtypeStruct((B, 1, 128), jnp.float32),
    )(obj.reshape(B, 8, _A // 8), fld.reshape(B, _NF, _NSEL), selk.reshape(B, 1, _NSEL),
      seli.reshape(B, 1, _NSEL), targets)

    box_loss = jnp.sum(out[:, 0, 0])
    cls_loss = jnp.sum(out[:, 0, 1])
    obj_loss = jnp.sum(out[:, 0, 2])
    total = box_loss + cls_loss + obj_loss
    return (total, box_loss, cls_loss, obj_loss)


# speedup vs baseline: 4.3395x; 1.0018x over previous
import functools

import jax
import jax.numpy as jnp
from jax import lax
from jax.experimental import pallas as pl
from jax.experimental.pallas import tpu as pltpu
from jax.experimental.pallas import tpu_sc as plsc

_A = 25200
_K = 300
_T = 20
_NCL = 4
_SPOS = 1.0 - 0.05
_SNEG = 0.05 / 3.0
_NCAND = 4096
_NV = _A // 16
_NSEL = 320
_NF = 9
_MIN32 = -2147483648
_M31 = 0x7FFFFFFF


def _key(x):
    b = lax.bitcast_convert_type(x, jnp.int32)
    return jnp.where(b >= 0, b, b ^ _M31)


def _popcnt(m):
    return plsc.all_reduce_population_count(m)


def _sc_body(obj_hbm, predf_hbm, fld_hbm, selk_hbm, seli_hbm,
             obj_v, ckey_v, cidx_v, selk_v, seli_v, gidx_v, fld_v, smem, sem):
    nc = 2
    wid = lax.axis_index("s") * nc + lax.axis_index("c")

    @pl.when(wid < 16)
    def _worker():
        s = wid
        pltpu.sync_copy(obj_hbm.at[pl.ds(s * 25600, _A)], obj_v)

        ones16 = jnp.ones((16,), jnp.int32)
        zeros16 = jnp.zeros((16,), jnp.int32)
        iot = lax.iota(jnp.int32, 16)

        def p1(i, carry):
            cntp, cnta, cntb = carry
            kk = _key(obj_v[pl.ds(i * 16, 16)])
            cntp = cntp + _popcnt(kk >= 0)
            cnta = cnta + _popcnt(kk >= jnp.int32(0x40000000))
            cntb = cntb + _popcnt(kk >= jnp.int32(-0x40000000))
            return cntp, cnta, cntb
        cntp, cnta, cntb = lax.fori_loop(0, _NV, p1, (zeros16, zeros16, zeros16))
        cnt31 = cntp[0]

        theta0 = jnp.where(cnt31 >= _K, _MIN32, jnp.int32(0))
        cnt0 = jnp.where(cnt31 >= _K, cnt31, jnp.int32(_A))
        cnt1 = jnp.where(cnt31 >= _K, cnta[0], cntb[0])
        take1 = cnt1 >= _K
        smem[0] = jnp.where(take1, theta0 | jnp.int32(1 << 30), theta0)
        smem[1] = jnp.int32(0)
        smem[2] = jnp.where(take1, cnt1, cnt0)

        def _count_pass(cand_i):
            def cp(i, acc):
                kk = _key(obj_v[pl.ds(i * 16, 16)])
                return acc + _popcnt(kk >= cand_i)
            return lax.fori_loop(0, _NV, cp, zeros16)[0]

        def _compact_pass():
            thr = smem[0] ^ _MIN32
            limit = jnp.full((16,), _NCAND, jnp.int32)
            minpad = jnp.full((16,), _MIN32, jnp.int32)
            def kp(i, off_v):
                kk = _key(obj_v[pl.ds(i * 16, 16)])
                msk = (kk >= thr) & (off_v < limit)
                csum = plsc.cumsum(jnp.where(msk, ones16, zeros16))
                pos = off_v + csum - 1
                plsc.store_scatter(ckey_v, [pos], kk, mask=msk)
                plsc.store_scatter(cidx_v, [pos], i * 16 + iot, mask=msk)
                return off_v + _popcnt(msk)
            off_v = lax.fori_loop(0, _NV, kp, zeros16)
            off = jnp.minimum(off_v[0], _NCAND)
            plsc.store_scatter(ckey_v, [off + iot], minpad)
            smem[3] = off
            smem[1] = jnp.int32(1)

        @pl.when(smem[2] <= _NCAND - 16)
        def _compact_now():
            _compact_pass()

        for lvl in range(2, 9):
            bit = jnp.int32(1 << (31 - lvl))

            @pl.when(smem[1] == 0)
            def _lvl():
                theta = smem[0]
                cnt = _count_pass((theta | bit) ^ _MIN32)
                take = cnt >= _K
                smem[0] = jnp.where(take, theta | bit, theta)
                smem[2] = jnp.where(take, cnt, smem[2])

            @pl.when((smem[1] == 0) & (smem[2] <= _NCAND - 16))
            def _do_compact():
                _compact_pass()

        @pl.when(smem[1] == 0)
        def _force_compact():
            _compact_pass()

        ncand = smem[3]
        nvec = (ncand + 15) // 16

        def bdesc(b, th):
            cand = th | (jnp.int32(1) << (jnp.int32(31) - b))
            cand_i = cand ^ _MIN32
            def cp(i, acc):
                return acc + _popcnt(ckey_v[pl.ds(i * 16, 16)] >= cand_i)
            cnt = lax.fori_loop(0, nvec, cp, zeros16)[0]
            return jnp.where(cnt >= _K, cand, th)
        theta2i = lax.fori_loop(0, 32, bdesc, jnp.int32(0)) ^ _MIN32

        kvec = jnp.full((16,), _K, jnp.int32)
        def selgt(i, off_v):
            kk = ckey_v[pl.ds(i * 16, 16)]
            ci = cidx_v[pl.ds(i * 16, 16)]
            msk = kk > theta2i
            csum = plsc.cumsum(jnp.where(msk, ones16, zeros16))
            pos = off_v + csum - 1
            plsc.store_scatter(selk_v, [pos], kk, mask=msk)
            plsc.store_scatter(seli_v, [pos], ci, mask=msk)
            return off_v + _popcnt(msk)
        c1_v = lax.fori_loop(0, nvec, selgt, zeros16)

        def seleq(i, off_v):
            kk = ckey_v[pl.ds(i * 16, 16)]
            ci = cidx_v[pl.ds(i * 16, 16)]
            msk = kk == theta2i
            csum = plsc.cumsum(jnp.where(msk, ones16, zeros16))
            pos = off_v + csum - 1
            wmsk = msk & (pos < kvec)
            plsc.store_scatter(selk_v, [pos], kk, mask=wmsk)
            plsc.store_scatter(seli_v, [pos], ci, mask=wmsk)
            return off_v + _popcnt(msk)
        lax.fori_loop(0, nvec, seleq, c1_v)

        pad = jnp.zeros((16,), jnp.int32)
        seli_v[pl.ds(304, 16)] = pad
        plsc.store_scatter(seli_v, [jnp.int32(_K) + iot], pad)
        selk_v[pl.ds(304, 16)] = pad
        plsc.store_scatter(selk_v, [jnp.int32(_K) + iot], pad)

        rowbase = jnp.int32(s * _A)
        for v in range(_NSEL // 16):
            base9 = (seli_v[pl.ds(v * 16, 16)] + rowbase) * 9
            for f in range(_NF):
                gidx_v[pl.ds(f * _NSEL + v * 16, 16)] = base9 + f
        cps = []
        for k in range(_NF * _NSEL // 96):
            cps.append(pltpu.async_copy(
                predf_hbm.at[gidx_v.at[pl.ds(k * 96, 96)]],
                fld_v.at[pl.ds(k * 96, 96)], sem))
        for cp in cps:
            cp.wait()

        pltpu.sync_copy(fld_v, fld_hbm.at[pl.ds(s * (_NF * _NSEL), _NF * _NSEL)])
        pltpu.sync_copy(selk_v.at[pl.ds(0, _NSEL)], selk_hbm.at[pl.ds(s * _NSEL, _NSEL)])
        pltpu.sync_copy(seli_v.at[pl.ds(0, _NSEL)], seli_hbm.at[pl.ds(s * _NSEL, _NSEL)])


def _softplus_tc(x):
    return jnp.maximum(x, 0.0) + jnp.log(1.0 + jnp.exp(-jnp.abs(x)))


def _tc_body(obj_ref, fld_ref, selk_ref, seli_ref, tgt_ref, out_ref):
    obj = obj_ref[...].reshape(200, 128)
    fld = fld_ref[...].reshape(_NF, _NSEL)
    selk = selk_ref[...].reshape(1, _NSEL)
    seli = seli_ref[...].reshape(1, _NSEL)
    tgt = tgt_ref[...].reshape(_T, 5)

    lane = lax.broadcasted_iota(jnp.int32, (1, _NSEL), 1)
    act = lane < _K
    zf = jnp.float32(0.0)
    big = jnp.int32(0x7FFFFFFF)

    bp0 = jax.nn.sigmoid(fld[0:1, :])
    bp1 = jax.nn.sigmoid(fld[1:2, :])
    bp2 = jax.nn.sigmoid(fld[2:3, :])
    bp3 = jax.nn.sigmoid(fld[3:4, :])
    ax1 = bp0 - bp2 * 0.5
    ay1 = bp1 - bp3 * 0.5
    ax2 = bp0 + bp2 * 0.5
    ay2 = bp1 + bp3 * 0.5
    a1 = (ax2 - ax1) * (ay2 - ay1)

    tcls = jnp.clip(tgt[:, 0:1].astype(jnp.int32), 0, _NCL - 1)
    tbc = [jnp.clip(tgt[:, 1 + c:2 + c], 0.0, 1.0) for c in range(4)]
    tx1 = tbc[0] - tbc[2] * 0.5
    ty1 = tbc[1] - tbc[3] * 0.5
    tx2 = tbc[0] + tbc[2] * 0.5
    ty2 = tbc[1] + tbc[3] * 0.5
    a2 = (tx2 - tx1) * (ty2 - ty1)

    iw = jnp.clip(jnp.minimum(ax2, tx2) - jnp.maximum(ax1, tx1), 0.0, None)
    ih = jnp.clip(jnp.minimum(ay2, ty2) - jnp.maximum(ay1, ty1), 0.0, None)
    inter = iw * ih
    union = a1 + a2 - inter
    iou = jnp.where(union > 0, inter / jnp.where(union > 0, union, 1.0), 0.0)
    iou_m = jnp.where(act, iou, jnp.float32(-1.0))

    bi = jnp.max(iou_m, axis=1, keepdims=True)
    e1 = iou_m == bi
    bk = jnp.max(jnp.where(e1, selk, jnp.int32(_MIN32)), axis=1, keepdims=True)
    e2 = e1 & (selk == bk)
    a_t = jnp.min(jnp.where(e2, seli, big), axis=1, keepdims=True)
    p_t = jnp.min(jnp.where(e2 & (seli == a_t), lane, big), axis=1, keepdims=True)

    a_row = jnp.transpose(a_t)
    same = a_t == a_row
    jj_col = lax.broadcasted_iota(jnp.int32, (_T, 1), 0)
    jj_row = lax.broadcasted_iota(jnp.int32, (1, _T), 1)
    is_first = ~jnp.any(same & (jj_row < jj_col), axis=1, keepdims=True)
    is_last = ~jnp.any(same & (jj_row > jj_col), axis=1, keepdims=True)
    cnt = jnp.sum(is_first.astype(jnp.float32))

    eqp = lane == p_t
    m = jnp.any(eqp, axis=0, keepdims=True)

    fi = lax.broadcasted_iota(jnp.int32, (200, 128), 0) * 128 + \
         lax.broadcasted_iota(jnp.int32, (200, 128), 1)
    sp_obj = jnp.where(fi < _A, _softplus_tc(obj), zf)
    dense = 0.5 * jnp.sum(sp_obj)
    objl_sel = fld[4:5, :]
    sp_sel = _softplus_tc(objl_sel)
    corr = jnp.sum(jnp.where(m, 1.5 * sp_sel - 2.0 * objl_sel, zf))
    obj_l = (dense + corr) / _A

    cls_sum = zf
    for c in range(4):
        row = fld[5 + c:6 + c, :]
        sp_row = _softplus_tc(row)
        cls_sum = cls_sum + jnp.sum(jnp.where(m, sp_row, zf))
        cls_sum = cls_sum - _SNEG * jnp.sum(jnp.where(m, row, zf))
        cm_c = jnp.minimum(
            jnp.sum(jnp.where(eqp & (tcls == c), 1.0, zf), axis=0, keepdims=True), 1.0)
        cls_sum = cls_sum - (_SPOS - _SNEG) * jnp.sum(cm_c * row)
    cls_l = cls_sum / (cnt * _NCL)

    bp = [bp0, bp1, bp2, bp3]
    box_sum = zf
    for c in range(4):
        bt_c = jnp.sum(jnp.where(eqp & is_last, tbc[c], zf), axis=0, keepdims=True)
        d = bp[c] - bt_c
        ad = jnp.abs(d)
        sl1 = jnp.where(ad < 0.1, 0.5 * d * d / 0.1, ad - 0.05)
        box_sum = box_sum + jnp.sum(jnp.where(m, sl1, zf))
    box_l = box_sum / (cnt * 4.0) * 2.0

    olane = lax.broadcasted_iota(jnp.int32, (1, 128), 1)
    vec = jnp.where(olane == 0, box_l,
          jnp.where(olane == 1, cls_l,
          jnp.where(olane == 2, obj_l, zf)))
    out_ref[...] = vec.reshape(out_ref.shape)


@jax.jit
def kernel(predictions, targets):
    B = predictions.shape[0]
    pred = predictions.reshape(B, _A, 9)
    _AP = 25600
    objp = jnp.pad(pred[:, :, 4], ((0, 0), (0, _AP - _A))).reshape(B * _AP)
    predf = predictions.reshape(B * _A * 9)

    mesh = plsc.VectorSubcoreMesh(core_axis_name="c", subcore_axis_name="s")
    f32 = jnp.float32
    i32 = jnp.int32

    @functools.partial(
        pl.kernel, mesh=mesh,
        out_type=(jax.ShapeDtypeStruct((B * _NF * _NSEL,), f32),
                  jax.ShapeDtypeStruct((B * _NSEL,), i32),
                  jax.ShapeDtypeStruct((B * _NSEL,), i32)),
        compiler_params=pltpu.CompilerParams(needs_layout_passes=False),
        scratch_types=[
            pltpu.VMEM((_A,), f32),
            pltpu.VMEM((_NCAND + 32,), i32),
            pltpu.VMEM((_NCAND + 32,), i32),
            pltpu.VMEM((_NSEL + 32,), i32),
            pltpu.VMEM((_NSEL + 32,), i32),
            pltpu.VMEM((_NF * _NSEL,), i32),
            pltpu.VMEM((_NF * _NSEL,), f32),
            pltpu.SMEM((8,), i32),
            pltpu.SemaphoreType.DMA,
        ],
    )
    def run(obj_hbm, predf_hbm, fld_hbm, selk_hbm, seli_hbm, *scratch):
        _sc_body(obj_hbm, predf_hbm, fld_hbm, selk_hbm, seli_hbm, *scratch)

    fld, selk, seli = run(objp, predf)

    out = pl.pallas_call(
        _tc_body,
        grid=(B,),
        in_specs=[
            pl.BlockSpec((25600,), lambda i: (i,)),
            pl.BlockSpec((1, _NF, _NSEL), lambda i: (i, 0, 0)),
            pl.BlockSpec((1, 1, _NSEL), lambda i: (i, 0, 0)),
            pl.BlockSpec((1, 1, _NSEL), lambda i: (i, 0, 0)),
            pl.BlockSpec((1, _T, 5), lambda i: (i, 0, 0)),
        ],
        out_specs=pl.BlockSpec((1, 1, 128), lambda i: (i, 0, 0)),
        out_shape=jax.ShapeDtypeStruct((B, 1, 128), jnp.float32),
    )(objp, fld.reshape(B, _NF, _NSEL), selk.reshape(B, 1, _NSEL),
      seli.reshape(B, 1, _NSEL), targets)

    box_loss = jnp.sum(out[:, 0, 0])
    cls_loss = jnp.sum(out[:, 0, 1])
    obj_loss = jnp.sum(out[:, 0, 2])
    total = box_loss + cls_loss + obj_loss
    return (total, box_loss, cls_loss, obj_loss)
